# Initial kernel scaffold; baseline (speedup 1.0000x reference)
import jax
import jax.numpy as jnp
from jax.experimental import pallas as pl


def kernel(wordid, posid, labelid, wordembed, posembed, labelembed, W_logits, b_logits):
    raise NotImplementedError("write your pallas kernel here")


# speedup vs baseline: 1.3316x; 1.3316x over previous
import functools

import jax
import jax.numpy as jnp
from jax import lax
from jax.experimental import pallas as pl
from jax.experimental.pallas import tpu as pltpu
from jax.experimental.pallas import tpu_sc as plsc

WORDDIM = 64
POSDIM = 32
NSLOT = 6
BATCH = 16384
TOTROWS = BATCH * NSLOT
NCORES = 2
NSUB = 16
NWORKERS = NCORES * NSUB
CHUNK = 128
ROWS_PER_W = TOTROWS // (NWORKERS * CHUNK)

BLK = 512
NBLK = BATCH // BLK


def _sc_gather(table, idx):
    mesh = plsc.VectorSubcoreMesh(
        core_axis_name="c", subcore_axis_name="s",
        num_cores=NCORES, num_subcores=NSUB)

    @functools.partial(
        pl.kernel,
        out_type=jax.ShapeDtypeStruct((TOTROWS, WORDDIM), jnp.float32),
        mesh=mesh,
        scratch_types=[
            pltpu.VMEM((ROWS_PER_W, CHUNK), jnp.int32),
            pltpu.VMEM((CHUNK, WORDDIM), jnp.float32),
            pltpu.VMEM((CHUNK, WORDDIM), jnp.float32),
            pltpu.SemaphoreType.DMA,
            pltpu.SemaphoreType.DMA,
            pltpu.SemaphoreType.DMA,
            pltpu.SemaphoreType.DMA,
        ],
        compiler_params=pltpu.CompilerParams(use_tc_tiling_on_sc=False),
    )
    def k(table_hbm, idx_hbm, out_hbm, idx_v, r0, r1, g0, g1, o0, o1):
        wid = lax.axis_index("s") * NCORES + lax.axis_index("c")
        base = wid * ROWS_PER_W
        pltpu.sync_copy(idx_hbm.at[pl.ds(base, ROWS_PER_W)], idx_v)
        bufs = (r0, r1)
        gsems = (g0, g1)
        osems = (o0, o1)
        gathers = [None, None]
        outs = [None, None]
        gathers[0] = pltpu.async_copy(table_hbm.at[idx_v.at[0]], r0, g0)
        for j in range(ROWS_PER_W):
            b = j % 2
            gathers[b].wait()
            if j + 1 < ROWS_PER_W:
                nb = (j + 1) % 2
                if outs[nb] is not None:
                    outs[nb].wait()
                gathers[nb] = pltpu.async_copy(
                    table_hbm.at[idx_v.at[j + 1]], bufs[nb], gsems[nb])
            outs[b] = pltpu.async_copy(
                bufs[b], out_hbm.at[pl.ds((base + j) * CHUNK, CHUNK)],
                osems[b])
        outs[ (ROWS_PER_W - 1) % 2 ].wait()
        if ROWS_PER_W >= 2:
            outs[ROWS_PER_W % 2].wait()

    return k(table, idx)


def _dense_body(g_ref, pos_ref, lab_ref, pe_ref, le_ref, w_ref, b_ref,
                out_ref):
    g = g_ref[...]
    cg = g * g * g
    w = w_ref[...]
    acc = jnp.dot(cg, w[: NSLOT * WORDDIM, :],
                  preferred_element_type=jnp.float32,
                  precision=lax.Precision.HIGHEST)
    pe = pe_ref[...]
    le = le_ref[...]
    pc = pe * pe * pe
    lc = le * le * le
    pos = pos_ref[0]
    lab = lab_ref[0]
    iota64 = lax.broadcasted_iota(jnp.int32, (BLK, 64), 1)
    off = NSLOT * WORDDIM
    for j in range(NSLOT):
        ptab = jnp.dot(pc, w[off + POSDIM * j: off + POSDIM * (j + 1), :],
                       preferred_element_type=jnp.float32,
                       precision=lax.Precision.HIGHEST)
        oh = (iota64 == pos[:, j:j + 1]).astype(jnp.float32)
        acc = acc + jnp.dot(oh, ptab, preferred_element_type=jnp.float32,
                            precision=lax.Precision.HIGHEST)
        ltab = jnp.dot(
            lc, w[off + NSLOT * POSDIM + POSDIM * j:
                  off + NSLOT * POSDIM + POSDIM * (j + 1), :],
            preferred_element_type=jnp.float32,
            precision=lax.Precision.HIGHEST)
        ohl = (iota64 == lab[:, j:j + 1]).astype(jnp.float32)
        acc = acc + jnp.dot(ohl, ltab, preferred_element_type=jnp.float32,
                            precision=lax.Precision.HIGHEST)
    out_ref[...] = acc + b_ref[0]


def _dense(gathered2d, posid3, labelid3, posembed, labelembed, w, b2):
    return pl.pallas_call(
        _dense_body,
        grid=(NBLK,),
        in_specs=[
            pl.BlockSpec((BLK, NSLOT * WORDDIM), lambda i: (i, 0)),
            pl.BlockSpec((1, BLK, NSLOT), lambda i: (i, 0, 0)),
            pl.BlockSpec((1, BLK, NSLOT), lambda i: (i, 0, 0)),
            pl.BlockSpec((64, POSDIM), lambda i: (0, 0)),
            pl.BlockSpec((64, POSDIM), lambda i: (0, 0)),
            pl.BlockSpec((768, 3), lambda i: (0, 0)),
            pl.BlockSpec((1, 3), lambda i: (0, 0)),
        ],
        out_specs=pl.BlockSpec((BLK, 3), lambda i: (i, 0)),
        out_shape=jax.ShapeDtypeStruct((BATCH, 3), jnp.float32),
    )(gathered2d, posid3, labelid3, posembed, labelembed, w, b2)


def kernel(wordid, posid, labelid, wordembed, posembed, labelembed,
           W_logits, b_logits):
    idx = wordid.astype(jnp.int32).reshape(NWORKERS * ROWS_PER_W, CHUNK)
    gathered = _sc_gather(wordembed, idx)
    g2 = gathered.reshape(BATCH, NSLOT * WORDDIM)
    out = _dense(
        g2,
        posid.astype(jnp.int32).reshape(NBLK, BLK, NSLOT),
        labelid.astype(jnp.int32).reshape(NBLK, BLK, NSLOT),
        posembed, labelembed, W_logits, b_logits.reshape(1, 3))
    return out


# speedup vs baseline: 1.5162x; 1.1386x over previous
import functools

import jax
import jax.numpy as jnp
from jax import lax
from jax.experimental import pallas as pl
from jax.experimental.pallas import tpu as pltpu
from jax.experimental.pallas import tpu_sc as plsc

WORDDIM = 64
POSDIM = 32
NSLOT = 6
BATCH = 16384
TOTROWS = BATCH * NSLOT
NCORES = 2
NSUB = 16
NWORKERS = NCORES * NSUB
CHUNK = 128
ROWS_PER_W = TOTROWS // (NWORKERS * CHUNK)

BLK = 512
NBLK = BATCH // BLK


def _sc_gather(table, idx):
    mesh = plsc.VectorSubcoreMesh(
        core_axis_name="c", subcore_axis_name="s",
        num_cores=NCORES, num_subcores=NSUB)

    @functools.partial(
        pl.kernel,
        out_type=jax.ShapeDtypeStruct((TOTROWS, WORDDIM), jnp.float32),
        mesh=mesh,
        scratch_types=[
            pltpu.VMEM((ROWS_PER_W, CHUNK), jnp.int32),
            pltpu.VMEM((CHUNK, WORDDIM), jnp.float32),
            pltpu.VMEM((CHUNK, WORDDIM), jnp.float32),
            pltpu.SemaphoreType.DMA,
            pltpu.SemaphoreType.DMA,
            pltpu.SemaphoreType.DMA,
            pltpu.SemaphoreType.DMA,
        ],
        compiler_params=pltpu.CompilerParams(use_tc_tiling_on_sc=False),
    )
    def k(table_hbm, idx_hbm, out_hbm, idx_v, r0, r1, g0, g1, o0, o1):
        wid = lax.axis_index("s") * NCORES + lax.axis_index("c")
        base = wid * ROWS_PER_W
        pltpu.sync_copy(idx_hbm.at[pl.ds(base, ROWS_PER_W)], idx_v)
        bufs = (r0, r1)
        gsems = (g0, g1)
        osems = (o0, o1)
        gathers = [None, None]
        outs = [None, None]
        gathers[0] = pltpu.async_copy(table_hbm.at[idx_v.at[0]], r0, g0)
        for j in range(ROWS_PER_W):
            b = j % 2
            gathers[b].wait()
            if j + 1 < ROWS_PER_W:
                nb = (j + 1) % 2
                if outs[nb] is not None:
                    outs[nb].wait()
                gathers[nb] = pltpu.async_copy(
                    table_hbm.at[idx_v.at[j + 1]], bufs[nb], gsems[nb])
            outs[b] = pltpu.async_copy(
                bufs[b], out_hbm.at[pl.ds((base + j) * CHUNK, CHUNK)],
                osems[b])
        outs[ (ROWS_PER_W - 1) % 2 ].wait()
        if ROWS_PER_W >= 2:
            outs[ROWS_PER_W % 2].wait()

    return k(table, idx)


def _dense_body(g_ref, pos_ref, lab_ref, pe_ref, le_ref, w_ref, b_ref,
                out_ref):
    g = g_ref[...]
    cg = g * g * g
    w = w_ref[...]
    acc = jnp.dot(cg, w[: NSLOT * WORDDIM, :],
                  preferred_element_type=jnp.float32)
    pe = pe_ref[...]
    le = le_ref[...]
    pc = pe * pe * pe
    lc = le * le * le
    pos = pos_ref[0]
    lab = lab_ref[0]
    iota64 = lax.broadcasted_iota(jnp.int32, (BLK, 64), 1)
    off = NSLOT * WORDDIM
    for j in range(NSLOT):
        ptab = jnp.dot(pc, w[off + POSDIM * j: off + POSDIM * (j + 1), :],
                       preferred_element_type=jnp.float32)
        oh = (iota64 == pos[:, j:j + 1]).astype(jnp.float32)
        acc = acc + jnp.dot(oh, ptab, preferred_element_type=jnp.float32)
        ltab = jnp.dot(
            lc, w[off + NSLOT * POSDIM + POSDIM * j:
                  off + NSLOT * POSDIM + POSDIM * (j + 1), :],
            preferred_element_type=jnp.float32)
        ohl = (iota64 == lab[:, j:j + 1]).astype(jnp.float32)
        acc = acc + jnp.dot(ohl, ltab, preferred_element_type=jnp.float32)
    out_ref[...] = acc + b_ref[0]


def _dense(gathered2d, posid3, labelid3, posembed, labelembed, w, b2):
    return pl.pallas_call(
        _dense_body,
        grid=(NBLK,),
        in_specs=[
            pl.BlockSpec((BLK, NSLOT * WORDDIM), lambda i: (i, 0)),
            pl.BlockSpec((1, BLK, NSLOT), lambda i: (i, 0, 0)),
            pl.BlockSpec((1, BLK, NSLOT), lambda i: (i, 0, 0)),
            pl.BlockSpec((64, POSDIM), lambda i: (0, 0)),
            pl.BlockSpec((64, POSDIM), lambda i: (0, 0)),
            pl.BlockSpec((768, 3), lambda i: (0, 0)),
            pl.BlockSpec((1, 3), lambda i: (0, 0)),
        ],
        out_specs=pl.BlockSpec((BLK, 3), lambda i: (i, 0)),
        out_shape=jax.ShapeDtypeStruct((BATCH, 3), jnp.float32),
    )(gathered2d, posid3, labelid3, posembed, labelembed, w, b2)


def kernel(wordid, posid, labelid, wordembed, posembed, labelembed,
           W_logits, b_logits):
    idx = wordid.astype(jnp.int32).reshape(NWORKERS * ROWS_PER_W, CHUNK)
    gathered = _sc_gather(wordembed, idx)
    g2 = gathered.reshape(BATCH, NSLOT * WORDDIM)
    out = _dense(
        g2,
        posid.astype(jnp.int32).reshape(NBLK, BLK, NSLOT),
        labelid.astype(jnp.int32).reshape(NBLK, BLK, NSLOT),
        posembed, labelembed, W_logits, b_logits.reshape(1, 3))
    return out


# speedup vs baseline: 1.7982x; 1.1860x over previous
import functools

import jax
import jax.numpy as jnp
from jax import lax
from jax.experimental import pallas as pl
from jax.experimental.pallas import tpu as pltpu
from jax.experimental.pallas import tpu_sc as plsc

WORDDIM = 64
POSDIM = 32
NSLOT = 6
BATCH = 16384
VOCAB = 1000000
TOTROWS = BATCH * NSLOT
NCORES = 2
NSUB = 16
NWORKERS = NCORES * NSUB
CHUNK = 128
ROWS_PER_W = TOTROWS // (NWORKERS * CHUNK)

RBLK = 2048
BLK = 512
NBLK = BATCH // BLK


def _repack_body(wt_ref, out_ref):
    a = wt_ref[...]
    at = a.T
    out_ref[:, :WORDDIM] = at
    out_ref[:, WORDDIM:] = jnp.zeros((RBLK, 128 - WORDDIM), jnp.float32)


def _repack(wt):
    return pl.pallas_call(
        _repack_body,
        grid=((VOCAB + RBLK - 1) // RBLK,),
        in_specs=[pl.BlockSpec((WORDDIM, RBLK), lambda i: (0, i))],
        out_specs=pl.BlockSpec((RBLK, 128), lambda i: (i, 0)),
        out_shape=jax.ShapeDtypeStruct((VOCAB, 128), jnp.float32),
    )(wt)


def _sc_gather(table128, idx):
    mesh = plsc.VectorSubcoreMesh(
        core_axis_name="c", subcore_axis_name="s",
        num_cores=NCORES, num_subcores=NSUB)

    @functools.partial(
        pl.kernel,
        out_type=jax.ShapeDtypeStruct((TOTROWS, WORDDIM), jnp.float32),
        mesh=mesh,
        scratch_types=[
            pltpu.VMEM((ROWS_PER_W, CHUNK), jnp.int32),
            pltpu.VMEM((CHUNK, 128), jnp.float32),
            pltpu.VMEM((CHUNK, 128), jnp.float32),
            pltpu.SemaphoreType.DMA,
            pltpu.SemaphoreType.DMA,
            pltpu.SemaphoreType.DMA,
            pltpu.SemaphoreType.DMA,
        ],
        compiler_params=pltpu.CompilerParams(use_tc_tiling_on_sc=False),
    )
    def k(table_hbm, idx_hbm, out_hbm, idx_v, r0, r1, g0, g1, o0, o1):
        wid = lax.axis_index("s") * NCORES + lax.axis_index("c")
        base = wid * ROWS_PER_W
        pltpu.sync_copy(idx_hbm.at[pl.ds(base, ROWS_PER_W)], idx_v)
        bufs = (r0, r1)
        gsems = (g0, g1)
        osems = (o0, o1)
        gathers = [None, None]
        outs = [None, None]
        gathers[0] = pltpu.async_copy(table_hbm.at[idx_v.at[0]], r0, g0)
        for j in range(ROWS_PER_W):
            b = j % 2
            gathers[b].wait()
            if j + 1 < ROWS_PER_W:
                nb = (j + 1) % 2
                if outs[nb] is not None:
                    outs[nb].wait()
                gathers[nb] = pltpu.async_copy(
                    table_hbm.at[idx_v.at[j + 1]], bufs[nb], gsems[nb])
            outs[b] = pltpu.async_copy(
                bufs[b].at[:, pl.ds(0, WORDDIM)],
                out_hbm.at[pl.ds((base + j) * CHUNK, CHUNK)],
                osems[b])
        outs[(ROWS_PER_W - 1) % 2].wait()
        outs[ROWS_PER_W % 2].wait()

    return k(table128, idx)


def _dense_body(g_ref, pos_ref, lab_ref, pe_ref, le_ref, w_ref, b_ref,
                out_ref):
    g = g_ref[...]
    cg = g * g * g
    w = w_ref[...]
    acc = jnp.dot(cg, w[: NSLOT * WORDDIM, :],
                  preferred_element_type=jnp.float32)
    pe = pe_ref[...]
    le = le_ref[...]
    pc = pe * pe * pe
    lc = le * le * le
    pos = pos_ref[0]
    lab = lab_ref[0]
    iota64 = lax.broadcasted_iota(jnp.int32, (BLK, 64), 1)
    off = NSLOT * WORDDIM
    for j in range(NSLOT):
        ptab = jnp.dot(pc, w[off + POSDIM * j: off + POSDIM * (j + 1), :],
                       preferred_element_type=jnp.float32)
        oh = (iota64 == pos[:, j:j + 1]).astype(jnp.float32)
        acc = acc + jnp.dot(oh, ptab, preferred_element_type=jnp.float32)
        ltab = jnp.dot(
            lc, w[off + NSLOT * POSDIM + POSDIM * j:
                  off + NSLOT * POSDIM + POSDIM * (j + 1), :],
            preferred_element_type=jnp.float32)
        ohl = (iota64 == lab[:, j:j + 1]).astype(jnp.float32)
        acc = acc + jnp.dot(ohl, ltab, preferred_element_type=jnp.float32)
    out_ref[...] = acc + b_ref[0]


def _dense(gathered2d, posid3, labelid3, posembed, labelembed, w, b2):
    return pl.pallas_call(
        _dense_body,
        grid=(NBLK,),
        in_specs=[
            pl.BlockSpec((BLK, NSLOT * WORDDIM), lambda i: (i, 0)),
            pl.BlockSpec((1, BLK, NSLOT), lambda i: (i, 0, 0)),
            pl.BlockSpec((1, BLK, NSLOT), lambda i: (i, 0, 0)),
            pl.BlockSpec((64, POSDIM), lambda i: (0, 0)),
            pl.BlockSpec((64, POSDIM), lambda i: (0, 0)),
            pl.BlockSpec((768, 3), lambda i: (0, 0)),
            pl.BlockSpec((1, 3), lambda i: (0, 0)),
        ],
        out_specs=pl.BlockSpec((BLK, 3), lambda i: (i, 0)),
        out_shape=jax.ShapeDtypeStruct((BATCH, 3), jnp.float32),
    )(gathered2d, posid3, labelid3, posembed, labelembed, w, b2)


def kernel(wordid, posid, labelid, wordembed, posembed, labelembed,
           W_logits, b_logits):
    table128 = _repack(wordembed.T)
    idx = wordid.astype(jnp.int32).reshape(NWORKERS * ROWS_PER_W, CHUNK)
    gathered = _sc_gather(table128, idx)
    g2 = gathered.reshape(BATCH, NSLOT * WORDDIM)
    out = _dense(
        g2,
        posid.astype(jnp.int32).reshape(NBLK, BLK, NSLOT),
        labelid.astype(jnp.int32).reshape(NBLK, BLK, NSLOT),
        posembed, labelembed, W_logits, b_logits.reshape(1, 3))
    return out
